# Initial kernel scaffold; baseline (speedup 1.0000x reference)
import jax
import jax.numpy as jnp
from jax.experimental import pallas as pl


def kernel(nfeat, efeat, ndist, edist, edge_index, W_gnn, b_gnn, W_ndist, b_ndist, W_edist, b_edist, W_nffn, b_nffn, W_effn, b_effn, W_nproj, b_nproj, W_eproj, b_eproj):
    raise NotImplementedError("write your pallas kernel here")


# speedup vs baseline: 2.2309x; 2.2309x over previous
import functools

import jax
import jax.numpy as jnp
from jax import lax
from jax.experimental import pallas as pl
from jax.experimental.pallas import tpu as pltpu
from jax.experimental.pallas import tpu_sc as plsc

N = 10000
E = 320000
D = 128
K = 32

NC = 2
NS = 16
NW = NC * NS
EPW = E // NW
CH = 80
NCHUNK = EPW // CH
NROWS_PT = 632
N_PAD = NROWS_PT * NS

_SC_MESH = plsc.VectorSubcoreMesh(core_axis_name="c", subcore_axis_name="s")


def _msg_agg_body(src_hbm, dst_hbm, nfeat_hbm, efeat_hbm, zeros_hbm, out_hbm,
                  src_v, dst_v, rows_v, ef_v, agg_sh, gsem):
    cid = lax.axis_index("c")
    sid = lax.axis_index("s")
    wid = sid * NC + cid
    pltpu.sync_copy(zeros_hbm, agg_sh.at[pl.ds(sid * NROWS_PT, NROWS_PT)])
    plsc.subcore_barrier()
    base = wid * EPW

    def chunk_body(c, carry):
        start = base + c * CH
        pltpu.sync_copy(src_hbm.at[pl.ds(start, CH)], src_v)
        pltpu.sync_copy(dst_hbm.at[pl.ds(start, CH)], dst_v)
        pltpu.async_copy(nfeat_hbm.at[src_v], rows_v, gsem).wait()
        pltpu.sync_copy(efeat_hbm.at[pl.ds(start, CH)], ef_v)

        def row_body(i, rcarry):
            for j in range(D // 16):
                sl = pl.ds(j * 16, 16)
                v = rows_v[i, sl] + ef_v[i, sl]
                rows_v[i, sl] = jnp.maximum(v, 0.0)
            return rcarry

        lax.fori_loop(0, CH, row_body, 0)
        pltpu.sync_copy(rows_v, agg_sh.at[dst_v], add=True)
        return carry

    lax.fori_loop(0, NCHUNK, chunk_body, 0)
    plsc.subcore_barrier()
    pltpu.sync_copy(agg_sh.at[pl.ds(sid * NROWS_PT, NROWS_PT)],
                    out_hbm.at[cid, pl.ds(sid * NROWS_PT, NROWS_PT)])


_msg_agg = functools.partial(
    pl.kernel,
    out_type=jax.ShapeDtypeStruct((NC, N_PAD, D), jnp.float32),
    mesh=_SC_MESH,
    scratch_types=[
        pltpu.VMEM((CH,), jnp.int32),
        pltpu.VMEM((CH,), jnp.int32),
        pltpu.VMEM((CH, D), jnp.float32),
        pltpu.VMEM((CH, D), jnp.float32),
        pltpu.VMEM_SHARED((N_PAD, D), jnp.float32),
        pltpu.SemaphoreType.DMA,
    ],
)(_msg_agg_body)


def _pair_gather_body(src_hbm, dst_hbm, amat_hbm, bmat_hbm, out_hbm,
                      src_v, dst_v, rows_a, rows_b, sema, semb):
    cid = lax.axis_index("c")
    sid = lax.axis_index("s")
    wid = sid * NC + cid
    base = wid * EPW

    def chunk_body(c, carry):
        start = base + c * CH
        pltpu.sync_copy(src_hbm.at[pl.ds(start, CH)], src_v)
        pltpu.sync_copy(dst_hbm.at[pl.ds(start, CH)], dst_v)
        cpa = pltpu.async_copy(amat_hbm.at[src_v], rows_a, sema)
        cpb = pltpu.async_copy(bmat_hbm.at[dst_v], rows_b, semb)
        cpa.wait()
        cpb.wait()

        def row_body(i, rcarry):
            for j in range(D // 16):
                sl = pl.ds(j * 16, 16)
                rows_a[i, sl] = rows_a[i, sl] + rows_b[i, sl]
            return rcarry

        lax.fori_loop(0, CH, row_body, 0)
        pltpu.sync_copy(rows_a, out_hbm.at[pl.ds(start, CH)])
        return carry

    lax.fori_loop(0, NCHUNK, chunk_body, 0)


_pair_gather = functools.partial(
    pl.kernel,
    out_type=jax.ShapeDtypeStruct((E, D), jnp.float32),
    mesh=_SC_MESH,
    scratch_types=[
        pltpu.VMEM((CH,), jnp.int32),
        pltpu.VMEM((CH,), jnp.int32),
        pltpu.VMEM((CH, D), jnp.float32),
        pltpu.VMEM((CH, D), jnp.float32),
        pltpu.SemaphoreType.DMA,
        pltpu.SemaphoreType.DMA,
    ],
)(_pair_gather_body)


def _node_body(nfeat_ref, aggp_ref, ndist_ref, wgnn_ref, bgnn_ref,
               wnd_ref, bnd_ref, wnf1_ref, wnf2_ref, bnf_ref,
               wnp_ref, bnp_ref, we1_ref, we2_ref,
               t_ref, p_ref, a_ref, b2_ref):
    f32 = jnp.float32
    xin = nfeat_ref[...] + aggp_ref[0] + aggp_ref[1]
    x = jnp.maximum(
        jnp.dot(xin, wgnn_ref[...], preferred_element_type=f32) + bgnn_ref[...],
        0.0)
    nd = jnp.maximum(
        jnp.dot(ndist_ref[...], wnd_ref[...], preferred_element_type=f32)
        + bnd_ref[...], 0.0)
    h = jnp.maximum(
        jnp.dot(x, wnf1_ref[...], preferred_element_type=f32)
        + jnp.dot(nd, wnf2_ref[...], preferred_element_type=f32)
        + bnf_ref[...], 0.0)
    o = jnp.dot(h, wnp_ref[...], preferred_element_type=f32) + bnp_ref[...]
    t_ref[...] = jnp.clip(o[:, 0:1], 1.0, 100.0)
    p_ref[...] = jax.nn.sigmoid(o[:, 1:2])
    a_ref[...] = jnp.dot(x, we1_ref[...], preferred_element_type=f32)
    b2_ref[...] = jnp.dot(x, we2_ref[...], preferred_element_type=f32)


def _edge_body(s_ref, edist_ref, wed_ref, bed_ref, we3_ref, bef_ref,
               wep_ref, bep_ref, t_ref, p_ref):
    f32 = jnp.float32
    ed = jnp.maximum(
        jnp.dot(edist_ref[...], wed_ref[...], preferred_element_type=f32)
        + bed_ref[...], 0.0)
    z = jnp.maximum(
        s_ref[...] + jnp.dot(ed, we3_ref[...], preferred_element_type=f32)
        + bef_ref[...], 0.0)
    o = jnp.dot(z, wep_ref[...], preferred_element_type=f32) + bep_ref[...]
    t_ref[...] = jnp.clip(o[:, 0:1], 1.0, 100.0)
    p_ref[...] = jax.nn.sigmoid(o[:, 1:2])


def _full_spec(shape):
    return pl.BlockSpec(shape, lambda i: (0,) * len(shape))


def kernel(nfeat, efeat, ndist, edist, edge_index,
           W_gnn, b_gnn, W_ndist, b_ndist, W_edist, b_edist,
           W_nffn, b_nffn, W_effn, b_effn,
           W_nproj, b_nproj, W_eproj, b_eproj):
    ei = edge_index.astype(jnp.int32)
    src = ei[0]
    dst = ei[1]
    zeros = jnp.zeros((NROWS_PT, D), jnp.float32)

    aggp = _msg_agg(src, dst, nfeat, efeat, zeros)

    wnf1 = W_nffn[:D]
    wnf2 = W_nffn[D:]
    we1 = W_effn[:D]
    we2 = W_effn[D:2 * D]
    we3 = W_effn[2 * D:]

    BN = 1000
    node_t, node_p, amat, bmat = pl.pallas_call(
        _node_body,
        grid=(N // BN,),
        in_specs=[
            pl.BlockSpec((BN, D), lambda i: (i, 0)),
            pl.BlockSpec((NC, BN, D), lambda i: (0, i, 0)),
            pl.BlockSpec((BN, K), lambda i: (i, 0)),
            _full_spec((D, D)),
            _full_spec((1, D)),
            _full_spec((K, K)),
            _full_spec((1, K)),
            _full_spec((D, D)),
            _full_spec((K, D)),
            _full_spec((1, D)),
            _full_spec((D, 2)),
            _full_spec((1, 2)),
            _full_spec((D, D)),
            _full_spec((D, D)),
        ],
        out_specs=[
            pl.BlockSpec((BN, 1), lambda i: (i, 0)),
            pl.BlockSpec((BN, 1), lambda i: (i, 0)),
            pl.BlockSpec((BN, D), lambda i: (i, 0)),
            pl.BlockSpec((BN, D), lambda i: (i, 0)),
        ],
        out_shape=[
            jax.ShapeDtypeStruct((N, 1), jnp.float32),
            jax.ShapeDtypeStruct((N, 1), jnp.float32),
            jax.ShapeDtypeStruct((N, D), jnp.float32),
            jax.ShapeDtypeStruct((N, D), jnp.float32),
        ],
    )(nfeat, aggp, ndist,
      W_gnn, b_gnn.reshape(1, D), W_ndist, b_ndist.reshape(1, K),
      wnf1, wnf2, b_nffn.reshape(1, D),
      W_nproj, b_nproj.reshape(1, 2), we1, we2)

    smat = _pair_gather(src, dst, amat, bmat)

    BE = 2000
    edge_t, edge_p = pl.pallas_call(
        _edge_body,
        grid=(E // BE,),
        in_specs=[
            pl.BlockSpec((BE, D), lambda i: (i, 0)),
            pl.BlockSpec((BE, K), lambda i: (i, 0)),
            _full_spec((K, K)),
            _full_spec((1, K)),
            _full_spec((K, D)),
            _full_spec((1, D)),
            _full_spec((D, 2)),
            _full_spec((1, 2)),
        ],
        out_specs=[
            pl.BlockSpec((BE, 1), lambda i: (i, 0)),
            pl.BlockSpec((BE, 1), lambda i: (i, 0)),
        ],
        out_shape=[
            jax.ShapeDtypeStruct((E, 1), jnp.float32),
            jax.ShapeDtypeStruct((E, 1), jnp.float32),
        ],
    )(smat, edist,
      W_edist, b_edist.reshape(1, K), we3, b_effn.reshape(1, D),
      W_eproj, b_eproj.reshape(1, 2))

    return (node_t, node_p, edge_t, edge_p)


# speedup vs baseline: 3.0284x; 1.3575x over previous
import functools

import jax
import jax.numpy as jnp
from jax import lax
from jax.experimental import pallas as pl
from jax.experimental.pallas import tpu as pltpu
from jax.experimental.pallas import tpu_sc as plsc

N = 10000
E = 320000
D = 128
K = 32

NC = 2
NS = 16
NW = NC * NS
EPW = E // NW
CH = 40
NCHUNK = EPW // CH
CHD = 80
NCHUNKD = EPW // CHD
NROWS_PT = 632
N_PAD = NROWS_PT * NS

_SC_MESH = plsc.VectorSubcoreMesh(core_axis_name="c", subcore_axis_name="s")


def _msg_agg_body(src_hbm, dst_hbm, nfeat_hbm, efeat_hbm, zeros_hbm, out_hbm,
                  src_v0, src_v1, sdst_v0, sdst_v1, rows_v0, rows_v1,
                  ef_v0, ef_v1, out_v0, out_v1, agg_sh,
                  gsem0, gsem1, esem0, esem1, ssem0, ssem1):
    cid = lax.axis_index("c")
    sid = lax.axis_index("s")
    wid = sid * NC + cid
    pltpu.sync_copy(zeros_hbm, agg_sh.at[pl.ds(sid * NROWS_PT, NROWS_PT)])
    plsc.subcore_barrier()
    base = wid * EPW

    bufs = ((src_v0, sdst_v0, rows_v0, ef_v0, out_v0, gsem0, esem0, ssem0),
            (src_v1, sdst_v1, rows_v1, ef_v1, out_v1, gsem1, esem1, ssem1))

    def issue(c, p):
        src_v, _, rows_v, ef_v, _, gsem, esem, _ = bufs[p]
        start = base + c * CH
        pltpu.sync_copy(src_hbm.at[pl.ds(start, CH)], src_v)
        pltpu.async_copy(nfeat_hbm.at[src_v], rows_v, gsem)
        pltpu.async_copy(efeat_hbm.at[pl.ds(start, CH)], ef_v, esem)

    def process(c, p, do_wait, do_issue):
        src_v, sdst_v, rows_v, ef_v, out_v, gsem, esem, ssem = bufs[p]
        start = base + c * CH
        if do_wait:
            pltpu.make_async_copy(out_v, agg_sh.at[sdst_v], ssem).wait()
        pltpu.sync_copy(dst_hbm.at[pl.ds(start, CH)], sdst_v)
        pltpu.make_async_copy(nfeat_hbm.at[src_v], rows_v, gsem).wait()
        pltpu.make_async_copy(efeat_hbm.at[pl.ds(start, CH)], ef_v, esem).wait()

        def row_body(i, rcarry):
            for j in range(D // 16):
                sl = pl.ds(j * 16, 16)
                out_v[i, sl] = jnp.maximum(rows_v[i, sl] + ef_v[i, sl], 0.0)
            return rcarry

        lax.fori_loop(0, CH, row_body, 0)
        pltpu.async_copy(out_v, agg_sh.at[sdst_v], ssem, add=True)
        if do_issue:
            issue(c + 2, p)

    issue(0, 0)
    issue(1, 1)
    process(0, 0, False, True)
    process(1, 1, False, True)

    def pair_body(t, carry):
        process(2 * t, 0, True, True)
        process(2 * t + 1, 1, True, True)
        return carry

    tmax = (NCHUNK - 4) // 2
    lax.fori_loop(1, tmax + 1, pair_body, 0)
    for c in range(2 * tmax + 2, NCHUNK):
        process(c, c % 2, True, c + 2 < NCHUNK)
    for p in (0, 1):
        _, sdst_v, _, _, out_v, _, _, ssem = bufs[p]
        pltpu.make_async_copy(out_v, agg_sh.at[sdst_v], ssem).wait()
    plsc.subcore_barrier()
    pltpu.sync_copy(agg_sh.at[pl.ds(sid * NROWS_PT, NROWS_PT)],
                    out_hbm.at[cid, pl.ds(sid * NROWS_PT, NROWS_PT)])


_msg_agg = functools.partial(
    pl.kernel,
    out_type=jax.ShapeDtypeStruct((NC, N_PAD, D), jnp.float32),
    mesh=_SC_MESH,
    scratch_types=[
        pltpu.VMEM((CH,), jnp.int32),
        pltpu.VMEM((CH,), jnp.int32),
        pltpu.VMEM((CH,), jnp.int32),
        pltpu.VMEM((CH,), jnp.int32),
        pltpu.VMEM((CH, D), jnp.float32),
        pltpu.VMEM((CH, D), jnp.float32),
        pltpu.VMEM((CH, D), jnp.float32),
        pltpu.VMEM((CH, D), jnp.float32),
        pltpu.VMEM((CH, D), jnp.float32),
        pltpu.VMEM((CH, D), jnp.float32),
        pltpu.VMEM_SHARED((N_PAD, D), jnp.float32),
        pltpu.SemaphoreType.DMA,
        pltpu.SemaphoreType.DMA,
        pltpu.SemaphoreType.DMA,
        pltpu.SemaphoreType.DMA,
        pltpu.SemaphoreType.DMA,
        pltpu.SemaphoreType.DMA,
    ],
)(_msg_agg_body)


def _pair_gather_body(src_hbm, dst_hbm, amat_hbm, bmat_hbm, out_hbm,
                      src_v0, src_v1, dst_v0, dst_v1, rows_a0, rows_a1,
                      rows_b0, rows_b1, out_v0, out_v1,
                      asem0, asem1, bsem0, bsem1, wsem0, wsem1):
    cid = lax.axis_index("c")
    sid = lax.axis_index("s")
    wid = sid * NC + cid
    base = wid * EPW

    bufs = ((src_v0, dst_v0, rows_a0, rows_b0, out_v0, asem0, bsem0, wsem0),
            (src_v1, dst_v1, rows_a1, rows_b1, out_v1, asem1, bsem1, wsem1))

    def issue(c, p):
        src_v, dst_v, rows_a, rows_b, _, asem, bsem, _ = bufs[p]
        start = base + c * CHD
        pltpu.sync_copy(src_hbm.at[pl.ds(start, CHD)], src_v)
        pltpu.sync_copy(dst_hbm.at[pl.ds(start, CHD)], dst_v)
        pltpu.async_copy(amat_hbm.at[src_v], rows_a, asem)
        pltpu.async_copy(bmat_hbm.at[dst_v], rows_b, bsem)

    def process(c, p, do_wait, do_issue):
        src_v, dst_v, rows_a, rows_b, out_v, asem, bsem, wsem = bufs[p]
        start = base + c * CHD
        pltpu.make_async_copy(amat_hbm.at[src_v], rows_a, asem).wait()
        pltpu.make_async_copy(bmat_hbm.at[dst_v], rows_b, bsem).wait()
        if do_wait:
            pltpu.make_async_copy(out_v, out_hbm.at[pl.ds(start, CHD)], wsem).wait()

        def row_body(i, rcarry):
            for j in range(D // 16):
                sl = pl.ds(j * 16, 16)
                out_v[i, sl] = rows_a[i, sl] + rows_b[i, sl]
            return rcarry

        lax.fori_loop(0, CHD, row_body, 0)
        pltpu.async_copy(out_v, out_hbm.at[pl.ds(start, CHD)], wsem)
        if do_issue:
            issue(c + 2, p)

    issue(0, 0)
    issue(1, 1)
    process(0, 0, False, True)
    process(1, 1, False, True)

    def pair_body(t, carry):
        process(2 * t, 0, True, True)
        process(2 * t + 1, 1, True, True)
        return carry

    tmax = (NCHUNKD - 4) // 2
    lax.fori_loop(1, tmax + 1, pair_body, 0)
    for c in range(2 * tmax + 2, NCHUNKD):
        process(c, c % 2, True, c + 2 < NCHUNKD)
    for p in (0, 1):
        _, _, _, _, out_v, _, _, wsem = bufs[p]
        pltpu.make_async_copy(out_v, out_hbm.at[pl.ds(0, CHD)], wsem).wait()


_pair_gather = functools.partial(
    pl.kernel,
    out_type=jax.ShapeDtypeStruct((E, D), jnp.float32),
    mesh=_SC_MESH,
    scratch_types=[
        pltpu.VMEM((CHD,), jnp.int32),
        pltpu.VMEM((CHD,), jnp.int32),
        pltpu.VMEM((CHD,), jnp.int32),
        pltpu.VMEM((CHD,), jnp.int32),
        pltpu.VMEM((CHD, D), jnp.float32),
        pltpu.VMEM((CHD, D), jnp.float32),
        pltpu.VMEM((CHD, D), jnp.float32),
        pltpu.VMEM((CHD, D), jnp.float32),
        pltpu.VMEM((CHD, D), jnp.float32),
        pltpu.VMEM((CHD, D), jnp.float32),
        pltpu.SemaphoreType.DMA,
        pltpu.SemaphoreType.DMA,
        pltpu.SemaphoreType.DMA,
        pltpu.SemaphoreType.DMA,
        pltpu.SemaphoreType.DMA,
        pltpu.SemaphoreType.DMA,
    ],
)(_pair_gather_body)


def _node_body(nfeat_ref, aggp_ref, ndist_ref, wgnn_ref, bgnn_ref,
               wnd_ref, bnd_ref, wnf1_ref, wnf2_ref, bnf_ref,
               wnp_ref, bnp_ref, we1_ref, we2_ref,
               t_ref, p_ref, a_ref, b2_ref):
    f32 = jnp.float32
    xin = nfeat_ref[...] + aggp_ref[0] + aggp_ref[1]
    x = jnp.maximum(
        jnp.dot(xin, wgnn_ref[...], preferred_element_type=f32) + bgnn_ref[...],
        0.0)
    nd = jnp.maximum(
        jnp.dot(ndist_ref[...], wnd_ref[...], preferred_element_type=f32)
        + bnd_ref[...], 0.0)
    h = jnp.maximum(
        jnp.dot(x, wnf1_ref[...], preferred_element_type=f32)
        + jnp.dot(nd, wnf2_ref[...], preferred_element_type=f32)
        + bnf_ref[...], 0.0)
    o = jnp.dot(h, wnp_ref[...], preferred_element_type=f32) + bnp_ref[...]
    t_ref[...] = jnp.clip(o[:, 0:1], 1.0, 100.0)
    p_ref[...] = jax.nn.sigmoid(o[:, 1:2])
    a_ref[...] = jnp.dot(x, we1_ref[...], preferred_element_type=f32)
    b2_ref[...] = jnp.dot(x, we2_ref[...], preferred_element_type=f32)


def _edge_body(s_ref, edist_ref, wed_ref, bed_ref, we3_ref, bef_ref,
               wep_ref, bep_ref, t_ref, p_ref):
    f32 = jnp.float32
    ed = jnp.maximum(
        jnp.dot(edist_ref[...], wed_ref[...], preferred_element_type=f32)
        + bed_ref[...], 0.0)
    z = jnp.maximum(
        s_ref[...] + jnp.dot(ed, we3_ref[...], preferred_element_type=f32)
        + bef_ref[...], 0.0)
    o = jnp.dot(z, wep_ref[...], preferred_element_type=f32) + bep_ref[...]
    t_ref[...] = jnp.clip(o[:, 0:1], 1.0, 100.0)
    p_ref[...] = jax.nn.sigmoid(o[:, 1:2])


def _full_spec(shape):
    return pl.BlockSpec(shape, lambda i: (0,) * len(shape))


def kernel(nfeat, efeat, ndist, edist, edge_index,
           W_gnn, b_gnn, W_ndist, b_ndist, W_edist, b_edist,
           W_nffn, b_nffn, W_effn, b_effn,
           W_nproj, b_nproj, W_eproj, b_eproj):
    ei = edge_index.astype(jnp.int32)
    src = ei[0]
    dst = ei[1]
    zeros = jnp.zeros((NROWS_PT, D), jnp.float32)

    aggp = _msg_agg(src, dst, nfeat, efeat, zeros)

    wnf1 = W_nffn[:D]
    wnf2 = W_nffn[D:]
    we1 = W_effn[:D]
    we2 = W_effn[D:2 * D]
    we3 = W_effn[2 * D:]

    BN = 1000
    node_t, node_p, amat, bmat = pl.pallas_call(
        _node_body,
        grid=(N // BN,),
        in_specs=[
            pl.BlockSpec((BN, D), lambda i: (i, 0)),
            pl.BlockSpec((NC, BN, D), lambda i: (0, i, 0)),
            pl.BlockSpec((BN, K), lambda i: (i, 0)),
            _full_spec((D, D)),
            _full_spec((1, D)),
            _full_spec((K, K)),
            _full_spec((1, K)),
            _full_spec((D, D)),
            _full_spec((K, D)),
            _full_spec((1, D)),
            _full_spec((D, 2)),
            _full_spec((1, 2)),
            _full_spec((D, D)),
            _full_spec((D, D)),
        ],
        out_specs=[
            pl.BlockSpec((BN, 1), lambda i: (i, 0)),
            pl.BlockSpec((BN, 1), lambda i: (i, 0)),
            pl.BlockSpec((BN, D), lambda i: (i, 0)),
            pl.BlockSpec((BN, D), lambda i: (i, 0)),
        ],
        out_shape=[
            jax.ShapeDtypeStruct((N, 1), jnp.float32),
            jax.ShapeDtypeStruct((N, 1), jnp.float32),
            jax.ShapeDtypeStruct((N, D), jnp.float32),
            jax.ShapeDtypeStruct((N, D), jnp.float32),
        ],
    )(nfeat, aggp, ndist,
      W_gnn, b_gnn.reshape(1, D), W_ndist, b_ndist.reshape(1, K),
      wnf1, wnf2, b_nffn.reshape(1, D),
      W_nproj, b_nproj.reshape(1, 2), we1, we2)

    smat = _pair_gather(src, dst, amat, bmat)

    BE = 2000
    edge_t, edge_p = pl.pallas_call(
        _edge_body,
        grid=(E // BE,),
        in_specs=[
            pl.BlockSpec((BE, D), lambda i: (i, 0)),
            pl.BlockSpec((BE, K), lambda i: (i, 0)),
            _full_spec((K, K)),
            _full_spec((1, K)),
            _full_spec((K, D)),
            _full_spec((1, D)),
            _full_spec((D, 2)),
            _full_spec((1, 2)),
        ],
        out_specs=[
            pl.BlockSpec((BE, 1), lambda i: (i, 0)),
            pl.BlockSpec((BE, 1), lambda i: (i, 0)),
        ],
        out_shape=[
            jax.ShapeDtypeStruct((E, 1), jnp.float32),
            jax.ShapeDtypeStruct((E, 1), jnp.float32),
        ],
    )(smat, edist,
      W_edist, b_edist.reshape(1, K), we3, b_effn.reshape(1, D),
      W_eproj, b_eproj.reshape(1, 2))

    return (node_t, node_p, edge_t, edge_p)
